# baseline (speedup 1.0000x reference)
import functools

import jax
import jax.numpy as jnp
from jax import lax
from jax.experimental import pallas as pl
from jax.experimental.pallas import tpu as pltpu
from jax.experimental.pallas import tpu_sc as plsc

N = 10000
E = 160000
D = 128
G = 4 * D

CHUNK = 128
BE = 1280
EA = 79360
EB = E - EA
ACC_ROWS = 10240
NGRP = 24


def _gather_call(table, src, dst, e0, ecnt):
    info = plsc.get_sparse_core_info()
    nc, ns = info.num_cores, info.num_subcores
    per_w = ecnt // 32
    nfull = per_w // CHUNK
    tail = per_w - nfull * CHUNK
    assert per_w % 8 == 0 and nfull % 2 == 1 and tail % 8 == 0 and 0 < tail
    mesh = plsc.VectorSubcoreMesh(core_axis_name="c", subcore_axis_name="s")

    @functools.partial(
        pl.kernel,
        mesh=mesh,
        out_type=(
            jax.ShapeDtypeStruct((ecnt, D), jnp.float32),
            jax.ShapeDtypeStruct((ecnt, D), jnp.float32),
        ),
        scratch_types=[
            pltpu.VMEM((per_w,), jnp.int32),
            pltpu.VMEM((per_w,), jnp.int32),
            pltpu.VMEM((2, CHUNK, D), jnp.float32),
            pltpu.VMEM((2, CHUNK, D), jnp.float32),
            pltpu.SemaphoreType.DMA,
            pltpu.SemaphoreType.DMA,
            pltpu.SemaphoreType.DMA,
            pltpu.SemaphoreType.DMA,
        ],
    )
    def gather_k(table_h, src_h, dst_h, out_s, out_d, idx_s, idx_d, buf_s,
                 buf_d, ss0, ss1, sd0, sd1):
        wid = lax.axis_index("s") * nc + lax.axis_index("c")
        base = wid * per_w
        pltpu.sync_copy(src_h.at[pl.ds(e0 + base, per_w)], idx_s)
        pltpu.sync_copy(dst_h.at[pl.ds(e0 + base, per_w)], idx_d)
        sems = ((ss0, sd0), (ss1, sd1))

        def descs(c, p, n):
            isl = pl.ds(c * CHUNK, n)
            bsl = pl.ds(0, n)
            ss, sd = sems[p]
            return ((table_h.at[idx_s.at[isl]], buf_s.at[p, bsl], ss),
                    (table_h.at[idx_d.at[isl]], buf_d.at[p, bsl], sd))

        def fire(c, p, n):
            for sref, dref, sem in descs(c, p, n):
                pltpu.async_copy(sref, dref, sem)

        def wait(c, p, n):
            for sref, dref, sem in descs(c, p, n):
                pltpu.make_async_copy(sref, dref, sem).wait()

        def store(c, p, n):
            off = base + c * CHUNK
            bsl = pl.ds(0, n)
            pltpu.sync_copy(buf_s.at[p, bsl], out_s.at[pl.ds(off, n)])
            pltpu.sync_copy(buf_d.at[p, bsl], out_d.at[pl.ds(off, n)])

        fire(0, 0, CHUNK)

        def body(gg, _):
            c0 = 2 * gg
            fire(c0 + 1, 1, CHUNK)
            wait(c0, 0, CHUNK)
            store(c0, 0, CHUNK)
            fire(c0 + 2, 0, CHUNK)
            wait(c0 + 1, 1, CHUNK)
            store(c0 + 1, 1, CHUNK)
            return _

        lax.fori_loop(0, (nfull - 1) // 2, body, None)
        fire(nfull, 1, tail)
        wait(nfull - 1, 0, CHUNK)
        store(nfull - 1, 0, CHUNK)
        wait(nfull, 1, tail)
        store(nfull, 1, tail)

    return gather_k(table, src, dst)


def _scatter_call(e_feat, idx3, zrows, e_local0, ecnt):
    info = plsc.get_sparse_core_info()
    nc, ns = info.num_cores, info.num_subcores
    per_w = ecnt // 32
    nfull = per_w // CHUNK
    tail = per_w - nfull * CHUNK
    assert per_w % 8 == 0 and nfull % 2 == 1 and tail % 8 == 0 and 0 < tail
    rows_per_tile = ACC_ROWS // ns
    mesh = plsc.VectorSubcoreMesh(core_axis_name="c", subcore_axis_name="s")

    @functools.partial(
        pl.kernel,
        mesh=mesh,
        out_type=jax.ShapeDtypeStruct((2, ACC_ROWS, D), jnp.float32),
        scratch_types=[
            pltpu.VMEM_SHARED((ACC_ROWS, D), jnp.float32),
            pltpu.VMEM((8, CHUNK), jnp.int32),
            pltpu.VMEM((2, CHUNK, D), jnp.float32),
            pltpu.SemaphoreType.DMA,
            pltpu.SemaphoreType.DMA,
        ],
    )
    def scatter_k(ef, idx3_h, zsrc, out, acc, idx2, buf, se0, se1):
        sid = lax.axis_index("s")
        cid = lax.axis_index("c")
        wid = sid * nc + cid
        base = e_local0 + wid * per_w
        tile0 = sid * rows_per_tile
        sems = (se0, se1)
        pltpu.sync_copy(zsrc, buf.at[0])

        def zloop(k, _):
            pltpu.sync_copy(buf.at[0],
                            acc.at[pl.ds(tile0 + k * CHUNK, CHUNK)])
            return _

        lax.fori_loop(0, rows_per_tile // CHUNK, zloop, None)
        plsc.subcore_barrier()

        def desc(c, p, n):
            return (ef.at[pl.ds(base + c * CHUNK, n)],
                    buf.at[p, pl.ds(0, n)], sems[p])

        def fire(c, p, n):
            sref, dref, sem = desc(c, p, n)
            pltpu.async_copy(sref, dref, sem)

        def wait(c, p, n):
            sref, dref, sem = desc(c, p, n)
            pltpu.make_async_copy(sref, dref, sem).wait()

        def scat(c, p):
            pltpu.sync_copy(buf.at[p], acc.at[idx2.at[lax.rem(c, 8)]],
                            add=True)

        fire(0, 0, CHUNK)

        def body(gg, _):
            c0 = 2 * gg
            fire(c0 + 1, 1, CHUNK)

            @pl.when(lax.rem(c0, 8) == 0)
            def _stage():
                pltpu.sync_copy(idx3_h.at[wid, pl.ds((c0 // 8) * 8, 8)], idx2)

            wait(c0, 0, CHUNK)
            scat(c0, 0)
            fire(c0 + 2, 0, CHUNK)
            wait(c0 + 1, 1, CHUNK)
            scat(c0 + 1, 1)
            return _

        lax.fori_loop(0, (nfull - 1) // 2, body, None)
        fire(nfull, 1, tail)
        wait(nfull - 1, 0, CHUNK)
        scat(nfull - 1, 0)
        wait(nfull, 1, tail)
        scat(nfull, 1)
        plsc.subcore_barrier()

        def xloop(k, _):
            pltpu.sync_copy(acc.at[pl.ds(tile0 + k * CHUNK, CHUNK)],
                            buf.at[0])
            pltpu.sync_copy(buf.at[0],
                            out.at[cid, pl.ds(tile0 + k * CHUNK, CHUNK)])
            return _

        lax.fori_loop(0, rows_per_tile // CHUNK, xloop, None)

    return scatter_k(e_feat, idx3, zrows)


def _make_edge_body(n_alias, with_efsc):
    def body(*refs):
        ef, sg, dg, eh, ec, gr, wx, ws, wd, wu, whh, bias = refs[:12]
        outs = refs[12 + n_alias:]
        if with_efsc:
            he_o, ce_o, eo_o, efsc_o, brow = outs
        else:
            he_o, ce_o, eo_o, brow = outs
        bf = jnp.bfloat16

        @pl.when(pl.program_id(0) == 0)
        def _():
            brow[...] = jnp.dot(gr[...].astype(bf), wu[...],
                                preferred_element_type=jnp.float32) \
                + bias[...]

        gates = jnp.dot(ef[...].astype(bf), wx[...],
                        preferred_element_type=jnp.float32)
        gates += jnp.dot(sg[...].astype(bf), ws[...],
                         preferred_element_type=jnp.float32)
        gates += jnp.dot(dg[...].astype(bf), wd[...],
                         preferred_element_type=jnp.float32)
        gates += jnp.dot(eh[...].astype(bf), whh[...],
                         preferred_element_type=jnp.float32)
        gates += brow[...]
        i = jax.nn.sigmoid(gates[:, :D])
        f = jax.nn.sigmoid(gates[:, D:2 * D])
        g = jnp.tanh(gates[:, 2 * D:3 * D])
        o = jax.nn.sigmoid(gates[:, 3 * D:])
        c_new = f * ec[...] + i * g
        h_new = o * jnp.tanh(c_new)
        relu = jnp.maximum(h_new, 0.0)
        he_o[...] = h_new
        ce_o[...] = c_new
        eo_o[...] = relu
        if with_efsc:
            efsc_o[...] = relu

    return body


def _edge_call(ef, sg, dg, eh2, ec2, g_repr, wx, ws, wd, wu, whh, bias,
               blk0, nblk, ecnt, aliased=None):
    off = lambda i: (i + blk0, 0)
    loc = lambda i: (i, 0)
    zero = lambda i: (0, 0)
    in_specs = [
        pl.BlockSpec((BE, D), off),
        pl.BlockSpec((BE, D), loc),
        pl.BlockSpec((BE, D), loc),
        pl.BlockSpec((BE, D), off),
        pl.BlockSpec((BE, D), off),
        pl.BlockSpec((1, D), zero),
        pl.BlockSpec((D, G), zero),
        pl.BlockSpec((D, G), zero),
        pl.BlockSpec((D, G), zero),
        pl.BlockSpec((D, G), zero),
        pl.BlockSpec((D, G), zero),
        pl.BlockSpec((1, G), zero),
    ]
    args = [ef, sg, dg, eh2, ec2, g_repr, wx, ws, wd, wu, whh, bias]
    out_specs = [pl.BlockSpec((BE, D), off)] * 3
    out_shape = [jax.ShapeDtypeStruct((E, D), jnp.float32)] * 3
    with_efsc = aliased is None
    if with_efsc:
        out_specs.append(pl.BlockSpec((BE, D), loc))
        out_shape.append(jax.ShapeDtypeStruct((ecnt, D), jnp.float32))
        io_aliases = {}
        n_alias = 0
    else:
        for a in aliased:
            in_specs.append(pl.BlockSpec(memory_space=pltpu.MemorySpace.HBM))
            args.append(a)
        io_aliases = {12: 0, 13: 1, 14: 2}
        n_alias = 3
    return pl.pallas_call(
        _make_edge_body(n_alias, with_efsc),
        grid=(nblk,),
        in_specs=in_specs,
        out_specs=out_specs,
        out_shape=out_shape,
        scratch_shapes=[pltpu.VMEM((1, G), jnp.float32)],
        input_output_aliases=io_aliases,
    )(*args)


BN = 1000


def _node_body(nf, pa, pb, nh, nc_, gr, wnx, wnm, wnu, wnhh, bn,
               gh, gc, wun, wue, wug, wuhh, bu,
               nf_o, hn_o, cn_o, uo_o, hu_o, cu_o, accn, acce):
    i_blk = pl.program_id(0)
    bf = jnp.bfloat16
    hm = (pa[0] + pa[1]) + (pb[0] + pb[1])
    gates = jnp.dot(nf[...].astype(bf), wnx[...],
                    preferred_element_type=jnp.float32)
    gates += jnp.dot(hm.astype(bf), wnm[...],
                     preferred_element_type=jnp.float32)
    gates += jnp.dot(nh[...].astype(bf), wnhh[...],
                     preferred_element_type=jnp.float32)
    gates += jnp.dot(gr[...].astype(bf), wnu[...],
                     preferred_element_type=jnp.float32) + bn[...]
    ig = jax.nn.sigmoid(gates[:, :D])
    fg = jax.nn.sigmoid(gates[:, D:2 * D])
    gg = jnp.tanh(gates[:, 2 * D:3 * D])
    og = jax.nn.sigmoid(gates[:, 3 * D:])
    c_new = fg * nc_[...] + ig * gg
    h_new = og * jnp.tanh(c_new)
    n_out = jnp.maximum(h_new, 0.0)
    nf_o[...] = n_out
    hn_o[...] = h_new
    cn_o[...] = c_new

    ns = jnp.sum(n_out, axis=0, keepdims=True)
    es = jnp.sum(hm, axis=0, keepdims=True)

    @pl.when(i_blk == 0)
    def _():
        accn[...] = ns
        acce[...] = es

    @pl.when(i_blk > 0)
    def _():
        accn[...] += ns
        acce[...] += es

    ug = jnp.dot(accn[...], wun[...], preferred_element_type=jnp.float32)
    ug += jnp.dot(acce[...], wue[...], preferred_element_type=jnp.float32)
    ug += jnp.dot(gr[...], wug[...], preferred_element_type=jnp.float32)
    ug += jnp.dot(gh[...], wuhh[...], preferred_element_type=jnp.float32)
    ug += bu[...]
    iu = jax.nn.sigmoid(ug[:, :D])
    fu = jax.nn.sigmoid(ug[:, D:2 * D])
    gu = jnp.tanh(ug[:, 2 * D:3 * D])
    ou = jax.nn.sigmoid(ug[:, 3 * D:])
    cu = fu * gc[...] + iu * gu
    hu = ou * jnp.tanh(cu)
    cu_o[...] = cu
    hu_o[...] = hu
    uo_o[...] = jnp.maximum(hu, 0.0)


def _node_call(nf, pa, pb, nh2, nc2, g_repr, wnx, wnm, wnu, wnhh, bn,
               gh2, gc2, wun, wue, wug, wuhh, bu):
    row = lambda i: (i, 0)
    zero = lambda i: (0, 0)
    prow = lambda i: (0, i, 0)
    return pl.pallas_call(
        _node_body,
        grid=(N // BN,),
        in_specs=[
            pl.BlockSpec((BN, D), row),
            pl.BlockSpec((2, BN, D), prow),
            pl.BlockSpec((2, BN, D), prow),
            pl.BlockSpec((BN, D), row),
            pl.BlockSpec((BN, D), row),
            pl.BlockSpec((1, D), zero),
            pl.BlockSpec((D, G), zero),
            pl.BlockSpec((D, G), zero),
            pl.BlockSpec((D, G), zero),
            pl.BlockSpec((D, G), zero),
            pl.BlockSpec((1, G), zero),
            pl.BlockSpec((1, D), zero),
            pl.BlockSpec((1, D), zero),
            pl.BlockSpec((D, G), zero),
            pl.BlockSpec((D, G), zero),
            pl.BlockSpec((D, G), zero),
            pl.BlockSpec((D, G), zero),
            pl.BlockSpec((1, G), zero),
        ],
        out_specs=[
            pl.BlockSpec((BN, D), row),
            pl.BlockSpec((BN, D), row),
            pl.BlockSpec((BN, D), row),
            pl.BlockSpec((1, D), zero),
            pl.BlockSpec((1, D), zero),
            pl.BlockSpec((1, D), zero),
        ],
        out_shape=[
            jax.ShapeDtypeStruct((N, D), jnp.float32),
            jax.ShapeDtypeStruct((N, D), jnp.float32),
            jax.ShapeDtypeStruct((N, D), jnp.float32),
            jax.ShapeDtypeStruct((1, D), jnp.float32),
            jax.ShapeDtypeStruct((1, D), jnp.float32),
            jax.ShapeDtypeStruct((1, D), jnp.float32),
        ],
        scratch_shapes=[
            pltpu.VMEM((1, D), jnp.float32),
            pltpu.VMEM((1, D), jnp.float32),
        ],
    )(nf, pa, pb, nh2, nc2, g_repr, wnx, wnm, wnu, wnhh, bn,
      gh2, gc2, wun, wue, wug, wuhh, bu)


def _pad_dst(dst_slice, per_w):
    arr = jnp.full((32, NGRP * CHUNK), N, jnp.int32)
    arr = arr.at[:, :per_w].set(dst_slice.reshape(32, per_w))
    return arr.reshape(32, NGRP, CHUNK)


def kernel(edge_index, edge_feat, node_feat, g_repr, edge_h, edge_c,
           node_h, node_c, graph_h, graph_c, W_ih_e, W_hh_e, b_ih_e, b_hh_e,
           W_ih_n, W_hh_n, b_ih_n, b_hh_n, W_ih_u, W_hh_u, b_ih_u, b_hh_u):
    src = edge_index[0].astype(jnp.int32)
    dst = edge_index[1].astype(jnp.int32)

    wte = W_ih_e.T.astype(jnp.bfloat16)
    we_x, we_s, we_d, we_u = (wte[:D], wte[D:2 * D], wte[2 * D:3 * D],
                              wte[3 * D:])
    whh_e = W_hh_e.T.astype(jnp.bfloat16)
    bias_e = (b_ih_e + b_hh_e).reshape(1, G).astype(jnp.float32)
    wtn = W_ih_n.T.astype(jnp.bfloat16)
    wn_x, wn_m, wn_u = wtn[:D], wtn[D:2 * D], wtn[2 * D:]
    whh_n = W_hh_n.T.astype(jnp.bfloat16)
    bias_n = (b_ih_n + b_hh_n).reshape(1, G).astype(jnp.float32)
    wtu = W_ih_u.T.astype(jnp.float32)
    wu_n, wu_e, wu_g = wtu[:D], wtu[D:2 * D], wtu[2 * D:]
    whh_u = W_hh_u.T.astype(jnp.float32)
    bias_u = (b_ih_u + b_hh_u).reshape(1, G).astype(jnp.float32)

    zrows = jnp.zeros((CHUNK, D), jnp.float32)
    dst_pad_a = _pad_dst(dst[:EA], EA // 32)
    dst_pad_b = _pad_dst(dst[EA:], EB // 32)

    sg_a, dg_a = _gather_call(node_feat, src, dst, 0, EA)
    sg_b, dg_b = _gather_call(node_feat, src, dst, EA, EB)

    he, ce, efull, efsc_a = _edge_call(
        edge_feat, sg_a, dg_a, edge_h[0], edge_c[0], g_repr, we_x, we_s,
        we_d, we_u, whh_e, bias_e, 0, EA // BE, EA)

    pa = _scatter_call(efsc_a, dst_pad_a, zrows, 0, EA)

    he, ce, efull = _edge_call(
        edge_feat, sg_b, dg_b, edge_h[0], edge_c[0], g_repr, we_x, we_s,
        we_d, we_u, whh_e, bias_e, EA // BE, EB // BE, EB,
        aliased=(he, ce, efull))

    pb = _scatter_call(efull, dst_pad_b, zrows, EA, EB)

    nf, hn, cn, u_out, hu, cu = _node_call(
        node_feat, pa, pb, node_h[0], node_c[0], g_repr, wn_x, wn_m, wn_u,
        whh_n, bias_n, graph_h[0], graph_c[0], wu_n, wu_e, wu_g, whh_u,
        bias_u)

    return (efull, he[None], ce[None], nf, hn[None], cn[None],
            u_out, hu[None], cu[None])


# speedup vs baseline: 1.0580x; 1.0580x over previous
import functools

import jax
import jax.numpy as jnp
from jax import lax
from jax.experimental import pallas as pl
from jax.experimental.pallas import tpu as pltpu
from jax.experimental.pallas import tpu_sc as plsc

N = 10000
E = 160000
D = 128
G = 4 * D

PER_W = E // 32
CHUNK = 128
NFULL = PER_W // CHUNK
TAIL = PER_W - NFULL * CHUNK
ACC_ROWS = 10240
EXP_H = 320


def _gather_call(table, src, dst, e0, ecnt):
    info = plsc.get_sparse_core_info()
    nc, ns = info.num_cores, info.num_subcores
    per_w = ecnt // 32
    nfull = per_w // CHUNK
    tail = per_w - nfull * CHUNK
    assert per_w % 8 == 0 and nfull % 2 == 1 and tail % 8 == 0 and 0 < tail
    mesh = plsc.VectorSubcoreMesh(core_axis_name="c", subcore_axis_name="s")

    @functools.partial(
        pl.kernel,
        mesh=mesh,
        out_type=(
            jax.ShapeDtypeStruct((ecnt, D), jnp.float32),
            jax.ShapeDtypeStruct((ecnt, D), jnp.float32),
        ),
        scratch_types=[
            pltpu.VMEM((per_w,), jnp.int32),
            pltpu.VMEM((per_w,), jnp.int32),
            pltpu.VMEM((2, CHUNK, D), jnp.float32),
            pltpu.VMEM((2, CHUNK, D), jnp.float32),
            pltpu.SemaphoreType.DMA,
            pltpu.SemaphoreType.DMA,
            pltpu.SemaphoreType.DMA,
            pltpu.SemaphoreType.DMA,
        ],
    )
    def gather_k(table, src_h, dst_h, out_s, out_d, idx_s, idx_d, buf_s,
                 buf_d, ss0, ss1, sd0, sd1):
        wid = lax.axis_index("s") * nc + lax.axis_index("c")
        base = wid * per_w
        pltpu.sync_copy(src_h.at[pl.ds(e0 + base, per_w)], idx_s)
        pltpu.sync_copy(dst_h.at[pl.ds(e0 + base, per_w)], idx_d)
        sems = ((ss0, sd0), (ss1, sd1))

        def descs(c, p, n):
            isl = pl.ds(c * CHUNK, n)
            bsl = pl.ds(0, n)
            ss, sd = sems[p]
            return ((table.at[idx_s.at[isl]], buf_s.at[p, bsl], ss),
                    (table.at[idx_d.at[isl]], buf_d.at[p, bsl], sd))

        def fire(c, p, n):
            for sref, dref, sem in descs(c, p, n):
                pltpu.async_copy(sref, dref, sem)

        def wait(c, p, n):
            for sref, dref, sem in descs(c, p, n):
                pltpu.make_async_copy(sref, dref, sem).wait()

        def store(c, p, n):
            off = base + c * CHUNK
            bsl = pl.ds(0, n)
            pltpu.sync_copy(buf_s.at[p, bsl], out_s.at[pl.ds(off, n)])
            pltpu.sync_copy(buf_d.at[p, bsl], out_d.at[pl.ds(off, n)])

        fire(0, 0, CHUNK)

        def body(gg, _):
            c0 = 2 * gg
            fire(c0 + 1, 1, CHUNK)
            wait(c0, 0, CHUNK)
            store(c0, 0, CHUNK)
            fire(c0 + 2, 0, CHUNK)
            wait(c0 + 1, 1, CHUNK)
            store(c0 + 1, 1, CHUNK)
            return _

        lax.fori_loop(0, (nfull - 1) // 2, body, None)
        fire(nfull, 1, tail)
        wait(nfull - 1, 0, CHUNK)
        store(nfull - 1, 0, CHUNK)
        wait(nfull, 1, tail)
        store(nfull, 1, tail)

    return gather_k(table, src, dst)


def _scatter_call(e_feat, dst_pad, zrows):
    info = plsc.get_sparse_core_info()
    nc, ns = info.num_cores, info.num_subcores
    rows_per_tile = ACC_ROWS // ns
    mesh = plsc.VectorSubcoreMesh(core_axis_name="c", subcore_axis_name="s")

    @functools.partial(
        pl.kernel,
        mesh=mesh,
        out_type=jax.ShapeDtypeStruct((2, ACC_ROWS, D), jnp.float32),
        scratch_types=[
            pltpu.VMEM_SHARED((ACC_ROWS, D), jnp.float32),
            pltpu.VMEM((8, CHUNK), jnp.int32),
            pltpu.VMEM((2, CHUNK, D), jnp.float32),
            pltpu.SemaphoreType.DMA,
            pltpu.SemaphoreType.DMA,
        ],
    )
    def scatter_k(ef, idx3, zsrc, out, acc, idx2, buf, se0, se1):
        sid = lax.axis_index("s")
        cid = lax.axis_index("c")
        wid = sid * nc + cid
        base = wid * PER_W
        tile0 = sid * rows_per_tile
        sems = (se0, se1)
        pltpu.sync_copy(zsrc, buf.at[0])

        def zloop(k, _):
            pltpu.sync_copy(buf.at[0],
                            acc.at[pl.ds(tile0 + k * CHUNK, CHUNK)])
            return _

        lax.fori_loop(0, rows_per_tile // CHUNK, zloop, None)
        plsc.subcore_barrier()

        def desc(c, p, n):
            return (ef.at[pl.ds(base + c * CHUNK, n)],
                    buf.at[p, pl.ds(0, n)], sems[p])

        def fire(c, p, n):
            sref, dref, sem = desc(c, p, n)
            pltpu.async_copy(sref, dref, sem)

        def wait(c, p, n):
            sref, dref, sem = desc(c, p, n)
            pltpu.make_async_copy(sref, dref, sem).wait()

        def scat(c, p):
            pltpu.sync_copy(buf.at[p], acc.at[idx2.at[lax.rem(c, 8)]],
                            add=True)

        fire(0, 0, CHUNK)

        def body(gg, _):
            c0 = 2 * gg
            fire(c0 + 1, 1, CHUNK)

            @pl.when(lax.rem(c0, 8) == 0)
            def _stage():
                pltpu.sync_copy(idx3.at[wid, pl.ds((c0 // 8) * 8, 8)], idx2)

            wait(c0, 0, CHUNK)
            scat(c0, 0)
            fire(c0 + 2, 0, CHUNK)
            wait(c0 + 1, 1, CHUNK)
            scat(c0 + 1, 1)
            return _

        lax.fori_loop(0, (NFULL - 1) // 2, body, None)
        fire(NFULL, 1, TAIL)
        wait(NFULL - 1, 0, CHUNK)
        scat(NFULL - 1, 0)
        wait(NFULL, 1, TAIL)
        scat(NFULL, 1)
        plsc.subcore_barrier()

        def xloop(k, _):
            pltpu.sync_copy(acc.at[pl.ds(tile0 + k * CHUNK, CHUNK)],
                            buf.at[0])
            pltpu.sync_copy(buf.at[0],
                            out.at[cid, pl.ds(tile0 + k * CHUNK, CHUNK)])
            return _

        lax.fori_loop(0, rows_per_tile // CHUNK, xloop, None)

    return scatter_k(e_feat, dst_pad, zrows)


BE = 1280
EA = 79360
EB = E - EA


def _make_edge_body(n_alias):
    def _edge_body(*refs):
        ef, sg, dg, eh, ec, gr, wx, ws, wd, wu, whh, bias = refs[:12]
        he_o, ce_o, eo_o, brow = refs[12 + n_alias:]
        bf = jnp.bfloat16

        @pl.when(pl.program_id(0) == 0)
        def _():
            brow[...] = jnp.dot(gr[...].astype(bf), wu[...],
                                preferred_element_type=jnp.float32) \
                + bias[...]

        gates = jnp.dot(ef[...].astype(bf), wx[...],
                        preferred_element_type=jnp.float32)
        gates += jnp.dot(sg[...].astype(bf), ws[...],
                         preferred_element_type=jnp.float32)
        gates += jnp.dot(dg[...].astype(bf), wd[...],
                         preferred_element_type=jnp.float32)
        gates += jnp.dot(eh[...].astype(bf), whh[...],
                         preferred_element_type=jnp.float32)
        gates += brow[...]
        i = jax.nn.sigmoid(gates[:, :D])
        f = jax.nn.sigmoid(gates[:, D:2 * D])
        g = jnp.tanh(gates[:, 2 * D:3 * D])
        o = jax.nn.sigmoid(gates[:, 3 * D:])
        c_new = f * ec[...] + i * g
        h_new = o * jnp.tanh(c_new)
        he_o[...] = h_new
        ce_o[...] = c_new
        eo_o[...] = jnp.maximum(h_new, 0.0)

    return _edge_body


def _edge_call(ef, sg, dg, eh2, ec2, g_repr, wx, ws, wd, wu, whh, bias,
               blk0, nblk, aliased=None):
    off = lambda i: (i + blk0, 0)
    loc = lambda i: (i, 0)
    zero = lambda i: (0, 0)
    in_specs = [
        pl.BlockSpec((BE, D), off),
        pl.BlockSpec((BE, D), loc),
        pl.BlockSpec((BE, D), loc),
        pl.BlockSpec((BE, D), off),
        pl.BlockSpec((BE, D), off),
        pl.BlockSpec((1, D), zero),
        pl.BlockSpec((D, G), zero),
        pl.BlockSpec((D, G), zero),
        pl.BlockSpec((D, G), zero),
        pl.BlockSpec((D, G), zero),
        pl.BlockSpec((D, G), zero),
        pl.BlockSpec((1, G), zero),
    ]
    args = [ef, sg, dg, eh2, ec2, g_repr, wx, ws, wd, wu, whh, bias]
    io_aliases = {}
    n_alias = 0
    if aliased is not None:
        for a in aliased:
            in_specs.append(pl.BlockSpec(memory_space=pltpu.MemorySpace.HBM))
            args.append(a)
        io_aliases = {12: 0, 13: 1, 14: 2}
        n_alias = 3
    return pl.pallas_call(
        _make_edge_body(n_alias),
        grid=(nblk,),
        in_specs=in_specs,
        out_specs=[pl.BlockSpec((BE, D), off)] * 3,
        out_shape=[jax.ShapeDtypeStruct((E, D), jnp.float32)] * 3,
        scratch_shapes=[pltpu.VMEM((1, G), jnp.float32)],
        input_output_aliases=io_aliases,
    )(*args)


BN = 1000


def _node_body(nf, p, nh, nc_, gr, wnx, wnm, wnu, wnhh, bn,
               gh, gc, wun, wue, wug, wuhh, bu,
               nf_o, hn_o, cn_o, uo_o, hu_o, cu_o, accn, acce):
    i_blk = pl.program_id(0)
    bf = jnp.bfloat16
    hm = p[0] + p[1]
    gates = jnp.dot(nf[...].astype(bf), wnx[...],
                    preferred_element_type=jnp.float32)
    gates += jnp.dot(hm.astype(bf), wnm[...],
                     preferred_element_type=jnp.float32)
    gates += jnp.dot(nh[...].astype(bf), wnhh[...],
                     preferred_element_type=jnp.float32)
    gates += jnp.dot(gr[...].astype(bf), wnu[...],
                     preferred_element_type=jnp.float32) + bn[...]
    ig = jax.nn.sigmoid(gates[:, :D])
    fg = jax.nn.sigmoid(gates[:, D:2 * D])
    gg = jnp.tanh(gates[:, 2 * D:3 * D])
    og = jax.nn.sigmoid(gates[:, 3 * D:])
    c_new = fg * nc_[...] + ig * gg
    h_new = og * jnp.tanh(c_new)
    n_out = jnp.maximum(h_new, 0.0)
    nf_o[...] = n_out
    hn_o[...] = h_new
    cn_o[...] = c_new

    ns = jnp.sum(n_out, axis=0, keepdims=True)
    es = jnp.sum(hm, axis=0, keepdims=True)

    @pl.when(i_blk == 0)
    def _():
        accn[...] = ns
        acce[...] = es

    @pl.when(i_blk > 0)
    def _():
        accn[...] += ns
        acce[...] += es

    ug = jnp.dot(accn[...], wun[...], preferred_element_type=jnp.float32)
    ug += jnp.dot(acce[...], wue[...], preferred_element_type=jnp.float32)
    ug += jnp.dot(gr[...], wug[...], preferred_element_type=jnp.float32)
    ug += jnp.dot(gh[...], wuhh[...], preferred_element_type=jnp.float32)
    ug += bu[...]
    iu = jax.nn.sigmoid(ug[:, :D])
    fu = jax.nn.sigmoid(ug[:, D:2 * D])
    gu = jnp.tanh(ug[:, 2 * D:3 * D])
    ou = jax.nn.sigmoid(ug[:, 3 * D:])
    cu = fu * gc[...] + iu * gu
    hu = ou * jnp.tanh(cu)
    cu_o[...] = cu
    hu_o[...] = hu
    uo_o[...] = jnp.maximum(hu, 0.0)


def _node_call(nf, partials, nh2, nc2, g_repr, wnx, wnm, wnu, wnhh, bn,
               gh2, gc2, wun, wue, wug, wuhh, bu):
    row = lambda i: (i, 0)
    zero = lambda i: (0, 0)
    zrow = lambda i: (0, 0)
    return pl.pallas_call(
        _node_body,
        grid=(N // BN,),
        in_specs=[
            pl.BlockSpec((BN, D), row),
            pl.BlockSpec((2, BN, D), lambda i: (0, i, 0)),
            pl.BlockSpec((BN, D), row),
            pl.BlockSpec((BN, D), row),
            pl.BlockSpec((1, D), zero),
            pl.BlockSpec((D, G), zero),
            pl.BlockSpec((D, G), zero),
            pl.BlockSpec((D, G), zero),
            pl.BlockSpec((D, G), zero),
            pl.BlockSpec((1, G), zero),
            pl.BlockSpec((1, D), zero),
            pl.BlockSpec((1, D), zero),
            pl.BlockSpec((D, G), zero),
            pl.BlockSpec((D, G), zero),
            pl.BlockSpec((D, G), zero),
            pl.BlockSpec((D, G), zero),
            pl.BlockSpec((1, G), zero),
        ],
        out_specs=[
            pl.BlockSpec((BN, D), row),
            pl.BlockSpec((BN, D), row),
            pl.BlockSpec((BN, D), row),
            pl.BlockSpec((1, D), zrow),
            pl.BlockSpec((1, D), zrow),
            pl.BlockSpec((1, D), zrow),
        ],
        out_shape=[
            jax.ShapeDtypeStruct((N, D), jnp.float32),
            jax.ShapeDtypeStruct((N, D), jnp.float32),
            jax.ShapeDtypeStruct((N, D), jnp.float32),
            jax.ShapeDtypeStruct((1, D), jnp.float32),
            jax.ShapeDtypeStruct((1, D), jnp.float32),
            jax.ShapeDtypeStruct((1, D), jnp.float32),
        ],
        scratch_shapes=[
            pltpu.VMEM((1, D), jnp.float32),
            pltpu.VMEM((1, D), jnp.float32),
        ],
    )(nf, partials, nh2, nc2, g_repr, wnx, wnm, wnu, wnhh, bn,
      gh2, gc2, wun, wue, wug, wuhh, bu)


def kernel(edge_index, edge_feat, node_feat, g_repr, edge_h, edge_c,
           node_h, node_c, graph_h, graph_c, W_ih_e, W_hh_e, b_ih_e, b_hh_e,
           W_ih_n, W_hh_n, b_ih_n, b_hh_n, W_ih_u, W_hh_u, b_ih_u, b_hh_u):
    src = edge_index[0].astype(jnp.int32)
    dst = edge_index[1].astype(jnp.int32)

    wte = W_ih_e.T.astype(jnp.bfloat16)
    we_x, we_s, we_d, we_u = (wte[:D], wte[D:2 * D], wte[2 * D:3 * D],
                              wte[3 * D:])
    whh_e = W_hh_e.T.astype(jnp.bfloat16)
    bias_e = (b_ih_e + b_hh_e).reshape(1, G).astype(jnp.float32)
    wtn = W_ih_n.T.astype(jnp.bfloat16)
    wn_x, wn_m, wn_u = wtn[:D], wtn[D:2 * D], wtn[2 * D:]
    whh_n = W_hh_n.T.astype(jnp.bfloat16)
    bias_n = (b_ih_n + b_hh_n).reshape(1, G).astype(jnp.float32)
    wtu = W_ih_u.T.astype(jnp.float32)
    wu_n, wu_e, wu_g = wtu[:D], wtu[D:2 * D], wtu[2 * D:]
    whh_u = W_hh_u.T.astype(jnp.float32)
    bias_u = (b_ih_u + b_hh_u).reshape(1, G).astype(jnp.float32)

    sg_a, dg_a = _gather_call(node_feat, src, dst, 0, EA)
    sg_b, dg_b = _gather_call(node_feat, src, dst, EA, EB)

    he, ce, e_feat = _edge_call(edge_feat, sg_a, dg_a, edge_h[0],
                                edge_c[0], g_repr, we_x, we_s, we_d, we_u,
                                whh_e, bias_e, 0, EA // BE)
    he, ce, e_feat = _edge_call(edge_feat, sg_b, dg_b, edge_h[0],
                                edge_c[0], g_repr, we_x, we_s, we_d, we_u,
                                whh_e, bias_e, EA // BE, EB // BE,
                                aliased=(he, ce, e_feat))

    dst_r = dst.reshape(32, PER_W)
    pad = jnp.full((32, CHUNK - TAIL), N, jnp.int32)
    dst_pad = jnp.concatenate([dst_r, pad], axis=1).reshape(32, NFULL + 1,
                                                            CHUNK)
    zrows = jnp.zeros((CHUNK, D), jnp.float32)
    partials = _scatter_call(e_feat, dst_pad, zrows)

    nf, hn, cn, u_out, hu, cu = _node_call(
        node_feat, partials, node_h[0], node_c[0], g_repr, wn_x, wn_m, wn_u,
        whh_n, bias_n, graph_h[0], graph_c[0], wu_n, wu_e, wu_g, whh_u,
        bias_u)

    return (e_feat, he[None], ce[None], nf, hn[None], cn[None],
            u_out, hu[None], cu[None])


# speedup vs baseline: 1.0597x; 1.0016x over previous
import functools

import jax
import jax.numpy as jnp
from jax import lax
from jax.experimental import pallas as pl
from jax.experimental.pallas import tpu as pltpu
from jax.experimental.pallas import tpu_sc as plsc

N = 10000
E = 160000
D = 128
G = 4 * D

PER_W = E // 32
CHUNK = 128
NFULL = PER_W // CHUNK
TAIL = PER_W - NFULL * CHUNK
ACC_ROWS = 10240
EXP_H = 320


def _gather_call(table, src, dst, e0, ecnt):
    info = plsc.get_sparse_core_info()
    nc, ns = info.num_cores, info.num_subcores
    per_w = ecnt // 32
    nfull = per_w // CHUNK
    tail = per_w - nfull * CHUNK
    assert per_w % 8 == 0 and nfull % 2 == 1 and tail % 8 == 0 and 0 < tail
    mesh = plsc.VectorSubcoreMesh(core_axis_name="c", subcore_axis_name="s")

    @functools.partial(
        pl.kernel,
        mesh=mesh,
        out_type=(
            jax.ShapeDtypeStruct((ecnt, D), jnp.float32),
            jax.ShapeDtypeStruct((ecnt, D), jnp.float32),
        ),
        scratch_types=[
            pltpu.VMEM((per_w,), jnp.int32),
            pltpu.VMEM((per_w,), jnp.int32),
            pltpu.VMEM((2, CHUNK, D), jnp.float32),
            pltpu.VMEM((2, CHUNK, D), jnp.float32),
            pltpu.SemaphoreType.DMA,
            pltpu.SemaphoreType.DMA,
            pltpu.SemaphoreType.DMA,
            pltpu.SemaphoreType.DMA,
        ],
    )
    def gather_k(table, src_h, dst_h, out_s, out_d, idx_s, idx_d, buf_s,
                 buf_d, ss0, ss1, sd0, sd1):
        wid = lax.axis_index("s") * nc + lax.axis_index("c")
        base = wid * per_w
        pltpu.sync_copy(src_h.at[pl.ds(e0 + base, per_w)], idx_s)
        pltpu.sync_copy(dst_h.at[pl.ds(e0 + base, per_w)], idx_d)
        sems = ((ss0, sd0), (ss1, sd1))

        def descs(c, p, n):
            isl = pl.ds(c * CHUNK, n)
            bsl = pl.ds(0, n)
            ss, sd = sems[p]
            return ((table.at[idx_s.at[isl]], buf_s.at[p, bsl], ss),
                    (table.at[idx_d.at[isl]], buf_d.at[p, bsl], sd))

        def fire(c, p, n):
            for sref, dref, sem in descs(c, p, n):
                pltpu.async_copy(sref, dref, sem)

        def wait(c, p, n):
            for sref, dref, sem in descs(c, p, n):
                pltpu.make_async_copy(sref, dref, sem).wait()

        def store(c, p, n):
            off = base + c * CHUNK
            bsl = pl.ds(0, n)
            pltpu.sync_copy(buf_s.at[p, bsl], out_s.at[pl.ds(off, n)])
            pltpu.sync_copy(buf_d.at[p, bsl], out_d.at[pl.ds(off, n)])

        fire(0, 0, CHUNK)

        def body(gg, _):
            c0 = 2 * gg
            fire(c0 + 1, 1, CHUNK)
            wait(c0, 0, CHUNK)
            store(c0, 0, CHUNK)
            fire(c0 + 2, 0, CHUNK)
            wait(c0 + 1, 1, CHUNK)
            store(c0 + 1, 1, CHUNK)
            return _

        lax.fori_loop(0, (nfull - 1) // 2, body, None)
        fire(nfull, 1, tail)
        wait(nfull - 1, 0, CHUNK)
        store(nfull - 1, 0, CHUNK)
        wait(nfull, 1, tail)
        store(nfull, 1, tail)

    return gather_k(table, src, dst)


def _scatter_call(e_feat, dst_pad, zrows):
    info = plsc.get_sparse_core_info()
    nc, ns = info.num_cores, info.num_subcores
    rows_per_tile = ACC_ROWS // ns
    mesh = plsc.VectorSubcoreMesh(core_axis_name="c", subcore_axis_name="s")

    @functools.partial(
        pl.kernel,
        mesh=mesh,
        out_type=jax.ShapeDtypeStruct((2, ACC_ROWS, D), jnp.float32),
        scratch_types=[
            pltpu.VMEM_SHARED((ACC_ROWS, D), jnp.float32),
            pltpu.VMEM((8, CHUNK), jnp.int32),
            pltpu.VMEM((2, CHUNK, D), jnp.float32),
            pltpu.SemaphoreType.DMA,
            pltpu.SemaphoreType.DMA,
        ],
    )
    def scatter_k(ef, idx3, zsrc, out, acc, idx2, buf, se0, se1):
        sid = lax.axis_index("s")
        cid = lax.axis_index("c")
        wid = sid * nc + cid
        base = wid * PER_W
        tile0 = sid * rows_per_tile
        sems = (se0, se1)
        pltpu.sync_copy(zsrc, buf.at[0])

        def zloop(k, _):
            pltpu.sync_copy(buf.at[0],
                            acc.at[pl.ds(tile0 + k * CHUNK, CHUNK)])
            return _

        lax.fori_loop(0, rows_per_tile // CHUNK, zloop, None)
        plsc.subcore_barrier()

        def desc(c, p, n):
            return (ef.at[pl.ds(base + c * CHUNK, n)],
                    buf.at[p, pl.ds(0, n)], sems[p])

        def fire(c, p, n):
            sref, dref, sem = desc(c, p, n)
            pltpu.async_copy(sref, dref, sem)

        def wait(c, p, n):
            sref, dref, sem = desc(c, p, n)
            pltpu.make_async_copy(sref, dref, sem).wait()

        def scat(c, p):
            pltpu.sync_copy(buf.at[p], acc.at[idx2.at[lax.rem(c, 8)]],
                            add=True)

        fire(0, 0, CHUNK)

        def body(gg, _):
            c0 = 2 * gg
            fire(c0 + 1, 1, CHUNK)

            @pl.when(lax.rem(c0, 8) == 0)
            def _stage():
                pltpu.sync_copy(idx3.at[wid, pl.ds((c0 // 8) * 8, 8)], idx2)

            wait(c0, 0, CHUNK)
            scat(c0, 0)
            fire(c0 + 2, 0, CHUNK)
            wait(c0 + 1, 1, CHUNK)
            scat(c0 + 1, 1)
            return _

        lax.fori_loop(0, (NFULL - 1) // 2, body, None)
        fire(NFULL, 1, TAIL)
        wait(NFULL - 1, 0, CHUNK)
        scat(NFULL - 1, 0)
        wait(NFULL, 1, TAIL)
        scat(NFULL, 1)
        plsc.subcore_barrier()

        def xloop(k, _):
            pltpu.sync_copy(acc.at[pl.ds(tile0 + k * CHUNK, CHUNK)],
                            buf.at[0])
            pltpu.sync_copy(buf.at[0],
                            out.at[cid, pl.ds(tile0 + k * CHUNK, CHUNK)])
            return _

        lax.fori_loop(0, rows_per_tile // CHUNK, xloop, None)

    return scatter_k(e_feat, dst_pad, zrows)


BE = 1280
EA = 79360
EB = E - EA


def _make_edge_body(n_alias):
    def _edge_body(*refs):
        ef, sg, dg, eh, ec, gr, wx, ws, wd, wu, whh, bias = refs[:12]
        he_o, ce_o, eo_o, brow = refs[12 + n_alias:]
        bf = jnp.bfloat16

        @pl.when(pl.program_id(0) == 0)
        def _():
            brow[...] = jnp.dot(gr[...].astype(bf), wu[...],
                                preferred_element_type=jnp.float32) \
                + bias[...]

        gates = jnp.dot(ef[...].astype(bf), wx[...],
                        preferred_element_type=jnp.float32)
        gates += jnp.dot(sg[...].astype(bf), ws[...],
                         preferred_element_type=jnp.float32)
        gates += jnp.dot(dg[...].astype(bf), wd[...],
                         preferred_element_type=jnp.float32)
        gates += jnp.dot(eh[...].astype(bf), whh[...],
                         preferred_element_type=jnp.float32)
        gates += brow[...]
        i = jax.nn.sigmoid(gates[:, :D])
        f = jax.nn.sigmoid(gates[:, D:2 * D])
        g = jnp.tanh(gates[:, 2 * D:3 * D])
        o = jax.nn.sigmoid(gates[:, 3 * D:])
        c_new = f * ec[...] + i * g
        h_new = o * jnp.tanh(c_new)
        he_o[...] = h_new
        ce_o[...] = c_new
        eo_o[...] = jnp.maximum(h_new, 0.0)

    return _edge_body


def _edge_call(ef, sg, dg, eh2, ec2, g_repr, wx, ws, wd, wu, whh, bias,
               blk0, nblk, aliased=None):
    off = lambda i: (i + blk0, 0)
    loc = lambda i: (i, 0)
    zero = lambda i: (0, 0)
    in_specs = [
        pl.BlockSpec((BE, D), off),
        pl.BlockSpec((BE, D), loc),
        pl.BlockSpec((BE, D), loc),
        pl.BlockSpec((BE, D), off),
        pl.BlockSpec((BE, D), off),
        pl.BlockSpec((1, D), zero),
        pl.BlockSpec((D, G), zero),
        pl.BlockSpec((D, G), zero),
        pl.BlockSpec((D, G), zero),
        pl.BlockSpec((D, G), zero),
        pl.BlockSpec((D, G), zero),
        pl.BlockSpec((1, G), zero),
    ]
    args = [ef, sg, dg, eh2, ec2, g_repr, wx, ws, wd, wu, whh, bias]
    io_aliases = {}
    n_alias = 0
    if aliased is not None:
        for a in aliased:
            in_specs.append(pl.BlockSpec(memory_space=pltpu.MemorySpace.HBM))
            args.append(a)
        io_aliases = {12: 0, 13: 1, 14: 2}
        n_alias = 3
    return pl.pallas_call(
        _make_edge_body(n_alias),
        grid=(nblk,),
        in_specs=in_specs,
        out_specs=[pl.BlockSpec((BE, D), off)] * 3,
        out_shape=[jax.ShapeDtypeStruct((E, D), jnp.float32)] * 3,
        scratch_shapes=[pltpu.VMEM((1, G), jnp.float32)],
        input_output_aliases=io_aliases,
    )(*args)


BN = 1000


def _node_body(nf, p, nh, nc_, gr, wnx, wnm, wnu, wnhh, bn,
               gh, gc, wun, wue, wug, wuhh, bu,
               nf_o, hn_o, cn_o, uo_o, hu_o, cu_o, accn, acce):
    i_blk = pl.program_id(0)
    bf = jnp.bfloat16
    hm = p[0] + p[1]
    gates = jnp.dot(nf[...].astype(bf), wnx[...],
                    preferred_element_type=jnp.float32)
    gates += jnp.dot(hm.astype(bf), wnm[...],
                     preferred_element_type=jnp.float32)
    gates += jnp.dot(nh[...].astype(bf), wnhh[...],
                     preferred_element_type=jnp.float32)
    gates += jnp.dot(gr[...].astype(bf), wnu[...],
                     preferred_element_type=jnp.float32) + bn[...]
    ig = jax.nn.sigmoid(gates[:, :D])
    fg = jax.nn.sigmoid(gates[:, D:2 * D])
    gg = jnp.tanh(gates[:, 2 * D:3 * D])
    og = jax.nn.sigmoid(gates[:, 3 * D:])
    c_new = fg * nc_[...] + ig * gg
    h_new = og * jnp.tanh(c_new)
    n_out = jnp.maximum(h_new, 0.0)
    nf_o[...] = n_out
    hn_o[...] = h_new
    cn_o[...] = c_new

    ns = jnp.sum(n_out, axis=0, keepdims=True)
    es = jnp.sum(hm, axis=0, keepdims=True)

    @pl.when(i_blk == 0)
    def _():
        accn[...] = ns
        acce[...] = es

    @pl.when(i_blk > 0)
    def _():
        accn[...] += ns
        acce[...] += es

    @pl.when(i_blk == pl.num_programs(0) - 1)
    def _():
        ug = jnp.dot(accn[...], wun[...],
                     preferred_element_type=jnp.float32)
        ug += jnp.dot(acce[...], wue[...],
                      preferred_element_type=jnp.float32)
        ug += jnp.dot(gr[...], wug[...], preferred_element_type=jnp.float32)
        ug += jnp.dot(gh[...], wuhh[...],
                      preferred_element_type=jnp.float32)
        ug += bu[...]
        iu = jax.nn.sigmoid(ug[:, :D])
        fu = jax.nn.sigmoid(ug[:, D:2 * D])
        gu = jnp.tanh(ug[:, 2 * D:3 * D])
        ou = jax.nn.sigmoid(ug[:, 3 * D:])
        cu = fu * gc[...] + iu * gu
        hu = ou * jnp.tanh(cu)
        cu_o[...] = cu
        hu_o[...] = hu
        uo_o[...] = jnp.maximum(hu, 0.0)


def _node_call(nf, partials, nh2, nc2, g_repr, wnx, wnm, wnu, wnhh, bn,
               gh2, gc2, wun, wue, wug, wuhh, bu):
    row = lambda i: (i, 0)
    zero = lambda i: (0, 0)
    zrow = lambda i: (0, 0)
    return pl.pallas_call(
        _node_body,
        grid=(N // BN,),
        in_specs=[
            pl.BlockSpec((BN, D), row),
            pl.BlockSpec((2, BN, D), lambda i: (0, i, 0)),
            pl.BlockSpec((BN, D), row),
            pl.BlockSpec((BN, D), row),
            pl.BlockSpec((1, D), zero),
            pl.BlockSpec((D, G), zero),
            pl.BlockSpec((D, G), zero),
            pl.BlockSpec((D, G), zero),
            pl.BlockSpec((D, G), zero),
            pl.BlockSpec((1, G), zero),
            pl.BlockSpec((1, D), zero),
            pl.BlockSpec((1, D), zero),
            pl.BlockSpec((D, G), zero),
            pl.BlockSpec((D, G), zero),
            pl.BlockSpec((D, G), zero),
            pl.BlockSpec((D, G), zero),
            pl.BlockSpec((1, G), zero),
        ],
        out_specs=[
            pl.BlockSpec((BN, D), row),
            pl.BlockSpec((BN, D), row),
            pl.BlockSpec((BN, D), row),
            pl.BlockSpec((1, D), zrow),
            pl.BlockSpec((1, D), zrow),
            pl.BlockSpec((1, D), zrow),
        ],
        out_shape=[
            jax.ShapeDtypeStruct((N, D), jnp.float32),
            jax.ShapeDtypeStruct((N, D), jnp.float32),
            jax.ShapeDtypeStruct((N, D), jnp.float32),
            jax.ShapeDtypeStruct((1, D), jnp.float32),
            jax.ShapeDtypeStruct((1, D), jnp.float32),
            jax.ShapeDtypeStruct((1, D), jnp.float32),
        ],
        scratch_shapes=[
            pltpu.VMEM((1, D), jnp.float32),
            pltpu.VMEM((1, D), jnp.float32),
        ],
    )(nf, partials, nh2, nc2, g_repr, wnx, wnm, wnu, wnhh, bn,
      gh2, gc2, wun, wue, wug, wuhh, bu)


def kernel(edge_index, edge_feat, node_feat, g_repr, edge_h, edge_c,
           node_h, node_c, graph_h, graph_c, W_ih_e, W_hh_e, b_ih_e, b_hh_e,
           W_ih_n, W_hh_n, b_ih_n, b_hh_n, W_ih_u, W_hh_u, b_ih_u, b_hh_u):
    src = edge_index[0].astype(jnp.int32)
    dst = edge_index[1].astype(jnp.int32)

    wte = W_ih_e.T.astype(jnp.bfloat16)
    we_x, we_s, we_d, we_u = (wte[:D], wte[D:2 * D], wte[2 * D:3 * D],
                              wte[3 * D:])
    whh_e = W_hh_e.T.astype(jnp.bfloat16)
    bias_e = (b_ih_e + b_hh_e).reshape(1, G).astype(jnp.float32)
    wtn = W_ih_n.T.astype(jnp.bfloat16)
    wn_x, wn_m, wn_u = wtn[:D], wtn[D:2 * D], wtn[2 * D:]
    whh_n = W_hh_n.T.astype(jnp.bfloat16)
    bias_n = (b_ih_n + b_hh_n).reshape(1, G).astype(jnp.float32)
    wtu = W_ih_u.T.astype(jnp.float32)
    wu_n, wu_e, wu_g = wtu[:D], wtu[D:2 * D], wtu[2 * D:]
    whh_u = W_hh_u.T.astype(jnp.float32)
    bias_u = (b_ih_u + b_hh_u).reshape(1, G).astype(jnp.float32)

    sg_a, dg_a = _gather_call(node_feat, src, dst, 0, EA)
    sg_b, dg_b = _gather_call(node_feat, src, dst, EA, EB)

    he, ce, e_feat = _edge_call(edge_feat, sg_a, dg_a, edge_h[0],
                                edge_c[0], g_repr, we_x, we_s, we_d, we_u,
                                whh_e, bias_e, 0, EA // BE)
    he, ce, e_feat = _edge_call(edge_feat, sg_b, dg_b, edge_h[0],
                                edge_c[0], g_repr, we_x, we_s, we_d, we_u,
                                whh_e, bias_e, EA // BE, EB // BE,
                                aliased=(he, ce, e_feat))

    dst_r = dst.reshape(32, PER_W)
    pad = jnp.full((32, CHUNK - TAIL), N, jnp.int32)
    dst_pad = jnp.concatenate([dst_r, pad], axis=1).reshape(32, NFULL + 1,
                                                            CHUNK)
    zrows = jnp.zeros((CHUNK, D), jnp.float32)
    partials = _scatter_call(e_feat, dst_pad, zrows)

    nf, hn, cn, u_out, hu, cu = _node_call(
        node_feat, partials, node_h[0], node_c[0], g_repr, wn_x, wn_m, wn_u,
        whh_n, bias_n, graph_h[0], graph_c[0], wu_n, wu_e, wu_g, whh_u,
        bias_u)

    return (e_feat, he[None], ce[None], nf, hn[None], cn[None],
            u_out, hu[None], cu[None])


# speedup vs baseline: 1.0707x; 1.0104x over previous
import functools

import jax
import jax.numpy as jnp
from jax import lax
from jax.experimental import pallas as pl
from jax.experimental.pallas import tpu as pltpu
from jax.experimental.pallas import tpu_sc as plsc

N = 10000
E = 160000
D = 128
G = 4 * D

PER_W = E // 32
CHUNK = 128
NFULL = PER_W // CHUNK
TAIL = PER_W - NFULL * CHUNK
ACC_ROWS = 10240
EXP_H = 320


def _gather_call(table, src, dst, e0, ecnt):
    info = plsc.get_sparse_core_info()
    nc, ns = info.num_cores, info.num_subcores
    per_w = ecnt // 32
    nfull = per_w // CHUNK
    tail = per_w - nfull * CHUNK
    assert per_w % 8 == 0 and nfull % 2 == 1 and tail % 8 == 0 and 0 < tail
    mesh = plsc.VectorSubcoreMesh(core_axis_name="c", subcore_axis_name="s")

    @functools.partial(
        pl.kernel,
        mesh=mesh,
        out_type=(
            jax.ShapeDtypeStruct((ecnt, D), jnp.float32),
            jax.ShapeDtypeStruct((ecnt, D), jnp.float32),
        ),
        scratch_types=[
            pltpu.VMEM((per_w,), jnp.int32),
            pltpu.VMEM((per_w,), jnp.int32),
            pltpu.VMEM((2, CHUNK, D), jnp.float32),
            pltpu.VMEM((2, CHUNK, D), jnp.float32),
            pltpu.SemaphoreType.DMA,
            pltpu.SemaphoreType.DMA,
            pltpu.SemaphoreType.DMA,
            pltpu.SemaphoreType.DMA,
        ],
    )
    def gather_k(table, src_h, dst_h, out_s, out_d, idx_s, idx_d, buf_s,
                 buf_d, ss0, ss1, sd0, sd1):
        wid = lax.axis_index("s") * nc + lax.axis_index("c")
        base = wid * per_w
        pltpu.sync_copy(src_h.at[pl.ds(e0 + base, per_w)], idx_s)
        pltpu.sync_copy(dst_h.at[pl.ds(e0 + base, per_w)], idx_d)
        sems = ((ss0, sd0), (ss1, sd1))

        def descs(c, p, n):
            isl = pl.ds(c * CHUNK, n)
            bsl = pl.ds(0, n)
            ss, sd = sems[p]
            return ((table.at[idx_s.at[isl]], buf_s.at[p, bsl], ss),
                    (table.at[idx_d.at[isl]], buf_d.at[p, bsl], sd))

        def fire(c, p, n):
            for sref, dref, sem in descs(c, p, n):
                pltpu.async_copy(sref, dref, sem)

        def wait(c, p, n):
            for sref, dref, sem in descs(c, p, n):
                pltpu.make_async_copy(sref, dref, sem).wait()

        def store(c, p, n):
            off = base + c * CHUNK
            bsl = pl.ds(0, n)
            pltpu.sync_copy(buf_s.at[p, bsl], out_s.at[pl.ds(off, n)])
            pltpu.sync_copy(buf_d.at[p, bsl], out_d.at[pl.ds(off, n)])

        fire(0, 0, CHUNK)

        def body(gg, _):
            c0 = 2 * gg
            fire(c0 + 1, 1, CHUNK)
            wait(c0, 0, CHUNK)
            store(c0, 0, CHUNK)
            fire(c0 + 2, 0, CHUNK)
            wait(c0 + 1, 1, CHUNK)
            store(c0 + 1, 1, CHUNK)
            return _

        lax.fori_loop(0, (nfull - 1) // 2, body, None)
        fire(nfull, 1, tail)
        wait(nfull - 1, 0, CHUNK)
        store(nfull - 1, 0, CHUNK)
        wait(nfull, 1, tail)
        store(nfull, 1, tail)

    return gather_k(table, src, dst)


def _scatter_call(e_feat, dst_pad, zrows):
    info = plsc.get_sparse_core_info()
    nc, ns = info.num_cores, info.num_subcores
    rows_per_tile = ACC_ROWS // ns
    mesh = plsc.VectorSubcoreMesh(core_axis_name="c", subcore_axis_name="s")

    @functools.partial(
        pl.kernel,
        mesh=mesh,
        out_type=jax.ShapeDtypeStruct((2, ACC_ROWS, D), jnp.float32),
        scratch_types=[
            pltpu.VMEM_SHARED((ACC_ROWS, D), jnp.float32),
            pltpu.VMEM((8, CHUNK), jnp.int32),
            pltpu.VMEM((2, CHUNK, D), jnp.float32),
            pltpu.SemaphoreType.DMA,
            pltpu.SemaphoreType.DMA,
        ],
    )
    def scatter_k(ef, idx3, zsrc, out, acc, idx2, buf, se0, se1):
        sid = lax.axis_index("s")
        cid = lax.axis_index("c")
        wid = sid * nc + cid
        base = wid * PER_W
        tile0 = sid * rows_per_tile
        sems = (se0, se1)
        pltpu.sync_copy(zsrc, buf.at[0])

        def zloop(k, _):
            pltpu.sync_copy(buf.at[0],
                            acc.at[pl.ds(tile0 + k * CHUNK, CHUNK)])
            return _

        lax.fori_loop(0, rows_per_tile // CHUNK, zloop, None)
        plsc.subcore_barrier()

        def desc(c, p, n):
            return (ef.at[pl.ds(base + c * CHUNK, n)],
                    buf.at[p, pl.ds(0, n)], sems[p])

        def fire(c, p, n):
            sref, dref, sem = desc(c, p, n)
            pltpu.async_copy(sref, dref, sem)

        def wait(c, p, n):
            sref, dref, sem = desc(c, p, n)
            pltpu.make_async_copy(sref, dref, sem).wait()

        def scat(c, p):
            pltpu.sync_copy(buf.at[p], acc.at[idx2.at[lax.rem(c, 8)]],
                            add=True)

        fire(0, 0, CHUNK)

        def body(gg, _):
            c0 = 2 * gg
            fire(c0 + 1, 1, CHUNK)

            @pl.when(lax.rem(c0, 8) == 0)
            def _stage():
                pltpu.sync_copy(idx3.at[wid, pl.ds((c0 // 8) * 8, 8)], idx2)

            wait(c0, 0, CHUNK)
            scat(c0, 0)
            fire(c0 + 2, 0, CHUNK)
            wait(c0 + 1, 1, CHUNK)
            scat(c0 + 1, 1)
            return _

        lax.fori_loop(0, (NFULL - 1) // 2, body, None)
        fire(NFULL, 1, TAIL)
        wait(NFULL - 1, 0, CHUNK)
        scat(NFULL - 1, 0)
        wait(NFULL, 1, TAIL)
        scat(NFULL, 1)
        plsc.subcore_barrier()

        def xloop(k, _):
            pltpu.sync_copy(acc.at[pl.ds(tile0 + k * CHUNK, CHUNK)],
                            buf.at[0])
            pltpu.sync_copy(buf.at[0],
                            out.at[cid, pl.ds(tile0 + k * CHUNK, CHUNK)])
            return _

        lax.fori_loop(0, rows_per_tile // CHUNK, xloop, None)

    return scatter_k(e_feat, dst_pad, zrows)


BE = 1280
E1 = 48640
E2 = 55040
E3 = 56320


def _make_edge_body(n_alias):
    def _edge_body(*refs):
        ef, sg, dg, eh, ec, gr, wx, ws, wd, wu, whh, bias = refs[:12]
        he_o, ce_o, eo_o, brow = refs[12 + n_alias:]
        bf = jnp.bfloat16

        @pl.when(pl.program_id(0) == 0)
        def _():
            brow[...] = jnp.dot(gr[...].astype(bf), wu[...],
                                preferred_element_type=jnp.float32) \
                + bias[...]

        gates = jnp.dot(ef[...].astype(bf), wx[...],
                        preferred_element_type=jnp.float32)
        gates += jnp.dot(sg[...].astype(bf), ws[...],
                         preferred_element_type=jnp.float32)
        gates += jnp.dot(dg[...].astype(bf), wd[...],
                         preferred_element_type=jnp.float32)
        gates += jnp.dot(eh[...].astype(bf), whh[...],
                         preferred_element_type=jnp.float32)
        gates += brow[...]
        i = jax.nn.sigmoid(gates[:, :D])
        f = jax.nn.sigmoid(gates[:, D:2 * D])
        g = jnp.tanh(gates[:, 2 * D:3 * D])
        o = jax.nn.sigmoid(gates[:, 3 * D:])
        c_new = f * ec[...] + i * g
        h_new = o * jnp.tanh(c_new)
        he_o[...] = h_new
        ce_o[...] = c_new
        eo_o[...] = jnp.maximum(h_new, 0.0)

    return _edge_body


def _edge_call(ef, sg, dg, eh2, ec2, g_repr, wx, ws, wd, wu, whh, bias,
               blk0, nblk, aliased=None):
    off = lambda i: (i + blk0, 0)
    loc = lambda i: (i, 0)
    zero = lambda i: (0, 0)
    in_specs = [
        pl.BlockSpec((BE, D), off),
        pl.BlockSpec((BE, D), loc),
        pl.BlockSpec((BE, D), loc),
        pl.BlockSpec((BE, D), off),
        pl.BlockSpec((BE, D), off),
        pl.BlockSpec((1, D), zero),
        pl.BlockSpec((D, G), zero),
        pl.BlockSpec((D, G), zero),
        pl.BlockSpec((D, G), zero),
        pl.BlockSpec((D, G), zero),
        pl.BlockSpec((D, G), zero),
        pl.BlockSpec((1, G), zero),
    ]
    args = [ef, sg, dg, eh2, ec2, g_repr, wx, ws, wd, wu, whh, bias]
    io_aliases = {}
    n_alias = 0
    if aliased is not None:
        for a in aliased:
            in_specs.append(pl.BlockSpec(memory_space=pltpu.MemorySpace.HBM))
            args.append(a)
        io_aliases = {12: 0, 13: 1, 14: 2}
        n_alias = 3
    return pl.pallas_call(
        _make_edge_body(n_alias),
        grid=(nblk,),
        in_specs=in_specs,
        out_specs=[pl.BlockSpec((BE, D), off)] * 3,
        out_shape=[jax.ShapeDtypeStruct((E, D), jnp.float32)] * 3,
        scratch_shapes=[pltpu.VMEM((1, G), jnp.float32)],
        input_output_aliases=io_aliases,
    )(*args)


BN = 1000


def _node_body(nf, p, nh, nc_, gr, wnx, wnm, wnu, wnhh, bn,
               gh, gc, wun, wue, wug, wuhh, bu,
               nf_o, hn_o, cn_o, uo_o, hu_o, cu_o, accn, acce):
    i_blk = pl.program_id(0)
    bf = jnp.bfloat16
    hm = p[0] + p[1]
    gates = jnp.dot(nf[...].astype(bf), wnx[...],
                    preferred_element_type=jnp.float32)
    gates += jnp.dot(hm.astype(bf), wnm[...],
                     preferred_element_type=jnp.float32)
    gates += jnp.dot(nh[...].astype(bf), wnhh[...],
                     preferred_element_type=jnp.float32)
    gates += jnp.dot(gr[...].astype(bf), wnu[...],
                     preferred_element_type=jnp.float32) + bn[...]
    ig = jax.nn.sigmoid(gates[:, :D])
    fg = jax.nn.sigmoid(gates[:, D:2 * D])
    gg = jnp.tanh(gates[:, 2 * D:3 * D])
    og = jax.nn.sigmoid(gates[:, 3 * D:])
    c_new = fg * nc_[...] + ig * gg
    h_new = og * jnp.tanh(c_new)
    n_out = jnp.maximum(h_new, 0.0)
    nf_o[...] = n_out
    hn_o[...] = h_new
    cn_o[...] = c_new

    ns = jnp.sum(n_out, axis=0, keepdims=True)
    es = jnp.sum(hm, axis=0, keepdims=True)

    @pl.when(i_blk == 0)
    def _():
        accn[...] = ns
        acce[...] = es

    @pl.when(i_blk > 0)
    def _():
        accn[...] += ns
        acce[...] += es

    @pl.when(i_blk == pl.num_programs(0) - 1)
    def _():
        ug = jnp.dot(accn[...], wun[...],
                     preferred_element_type=jnp.float32)
        ug += jnp.dot(acce[...], wue[...],
                      preferred_element_type=jnp.float32)
        ug += jnp.dot(gr[...], wug[...], preferred_element_type=jnp.float32)
        ug += jnp.dot(gh[...], wuhh[...],
                      preferred_element_type=jnp.float32)
        ug += bu[...]
        iu = jax.nn.sigmoid(ug[:, :D])
        fu = jax.nn.sigmoid(ug[:, D:2 * D])
        gu = jnp.tanh(ug[:, 2 * D:3 * D])
        ou = jax.nn.sigmoid(ug[:, 3 * D:])
        cu = fu * gc[...] + iu * gu
        hu = ou * jnp.tanh(cu)
        cu_o[...] = cu
        hu_o[...] = hu
        uo_o[...] = jnp.maximum(hu, 0.0)


def _node_call(nf, partials, nh2, nc2, g_repr, wnx, wnm, wnu, wnhh, bn,
               gh2, gc2, wun, wue, wug, wuhh, bu):
    row = lambda i: (i, 0)
    zero = lambda i: (0, 0)
    zrow = lambda i: (0, 0)
    return pl.pallas_call(
        _node_body,
        grid=(N // BN,),
        in_specs=[
            pl.BlockSpec((BN, D), row),
            pl.BlockSpec((2, BN, D), lambda i: (0, i, 0)),
            pl.BlockSpec((BN, D), row),
            pl.BlockSpec((BN, D), row),
            pl.BlockSpec((1, D), zero),
            pl.BlockSpec((D, G), zero),
            pl.BlockSpec((D, G), zero),
            pl.BlockSpec((D, G), zero),
            pl.BlockSpec((D, G), zero),
            pl.BlockSpec((1, G), zero),
            pl.BlockSpec((1, D), zero),
            pl.BlockSpec((1, D), zero),
            pl.BlockSpec((D, G), zero),
            pl.BlockSpec((D, G), zero),
            pl.BlockSpec((D, G), zero),
            pl.BlockSpec((D, G), zero),
            pl.BlockSpec((1, G), zero),
        ],
        out_specs=[
            pl.BlockSpec((BN, D), row),
            pl.BlockSpec((BN, D), row),
            pl.BlockSpec((BN, D), row),
            pl.BlockSpec((1, D), zrow),
            pl.BlockSpec((1, D), zrow),
            pl.BlockSpec((1, D), zrow),
        ],
        out_shape=[
            jax.ShapeDtypeStruct((N, D), jnp.float32),
            jax.ShapeDtypeStruct((N, D), jnp.float32),
            jax.ShapeDtypeStruct((N, D), jnp.float32),
            jax.ShapeDtypeStruct((1, D), jnp.float32),
            jax.ShapeDtypeStruct((1, D), jnp.float32),
            jax.ShapeDtypeStruct((1, D), jnp.float32),
        ],
        scratch_shapes=[
            pltpu.VMEM((1, D), jnp.float32),
            pltpu.VMEM((1, D), jnp.float32),
        ],
    )(nf, partials, nh2, nc2, g_repr, wnx, wnm, wnu, wnhh, bn,
      gh2, gc2, wun, wue, wug, wuhh, bu)


def kernel(edge_index, edge_feat, node_feat, g_repr, edge_h, edge_c,
           node_h, node_c, graph_h, graph_c, W_ih_e, W_hh_e, b_ih_e, b_hh_e,
           W_ih_n, W_hh_n, b_ih_n, b_hh_n, W_ih_u, W_hh_u, b_ih_u, b_hh_u):
    src = edge_index[0].astype(jnp.int32)
    dst = edge_index[1].astype(jnp.int32)

    wte = W_ih_e.T.astype(jnp.bfloat16)
    we_x, we_s, we_d, we_u = (wte[:D], wte[D:2 * D], wte[2 * D:3 * D],
                              wte[3 * D:])
    whh_e = W_hh_e.T.astype(jnp.bfloat16)
    bias_e = (b_ih_e + b_hh_e).reshape(1, G).astype(jnp.float32)
    wtn = W_ih_n.T.astype(jnp.bfloat16)
    wn_x, wn_m, wn_u = wtn[:D], wtn[D:2 * D], wtn[2 * D:]
    whh_n = W_hh_n.T.astype(jnp.bfloat16)
    bias_n = (b_ih_n + b_hh_n).reshape(1, G).astype(jnp.float32)
    wtu = W_ih_u.T.astype(jnp.float32)
    wu_n, wu_e, wu_g = wtu[:D], wtu[D:2 * D], wtu[2 * D:]
    whh_u = W_hh_u.T.astype(jnp.float32)
    bias_u = (b_ih_u + b_hh_u).reshape(1, G).astype(jnp.float32)

    sg_a, dg_a = _gather_call(node_feat, src, dst, 0, E1)
    sg_b, dg_b = _gather_call(node_feat, src, dst, E1, E2)
    sg_c, dg_c = _gather_call(node_feat, src, dst, E1 + E2, E3)

    he, ce, e_feat = _edge_call(edge_feat, sg_a, dg_a, edge_h[0],
                                edge_c[0], g_repr, we_x, we_s, we_d, we_u,
                                whh_e, bias_e, 0, E1 // BE)
    he, ce, e_feat = _edge_call(edge_feat, sg_b, dg_b, edge_h[0],
                                edge_c[0], g_repr, we_x, we_s, we_d, we_u,
                                whh_e, bias_e, E1 // BE, E2 // BE,
                                aliased=(he, ce, e_feat))
    he, ce, e_feat = _edge_call(edge_feat, sg_c, dg_c, edge_h[0],
                                edge_c[0], g_repr, we_x, we_s, we_d, we_u,
                                whh_e, bias_e, (E1 + E2) // BE, E3 // BE,
                                aliased=(he, ce, e_feat))

    dst_r = dst.reshape(32, PER_W)
    pad = jnp.full((32, CHUNK - TAIL), N, jnp.int32)
    dst_pad = jnp.concatenate([dst_r, pad], axis=1).reshape(32, NFULL + 1,
                                                            CHUNK)
    zrows = jnp.zeros((CHUNK, D), jnp.float32)
    partials = _scatter_call(e_feat, dst_pad, zrows)

    nf, hn, cn, u_out, hu, cu = _node_call(
        node_feat, partials, node_h[0], node_c[0], g_repr, wn_x, wn_m, wn_u,
        whh_n, bias_n, graph_h[0], graph_c[0], wu_n, wu_e, wu_g, whh_u,
        bias_u)

    return (e_feat, he[None], ce[None], nf, hn[None], cn[None],
            u_out, hu[None], cu[None])
